# baseline (speedup 1.0000x reference)
import functools

import jax
import jax.numpy as jnp
from jax import lax
from jax.experimental import pallas as pl
from jax.experimental.pallas import tpu as pltpu
from jax.experimental.pallas import tpu_sc as plsc


def kernel(input, indices):
    B, C, H, W = input.shape
    P = B * C
    Hout, Wout = 2 * H, 2 * W

    info = plsc.get_sparse_core_info()
    NC, NS = info.num_cores, info.num_subcores
    NW = NC * NS

    planes_per_w = P // NW
    RH = 48
    S = H // RH
    IN_STRIP = RH * W
    OUT_STRIP = 2 * RH * Wout
    NSTRIPS = planes_per_w * S

    in_flat = input.reshape(P * H * W)
    idx_flat = indices.reshape(P * H * W)

    mesh = plsc.VectorSubcoreMesh(core_axis_name="c", subcore_axis_name="s")

    @functools.partial(
        pl.kernel,
        mesh=mesh,
        out_type=jax.ShapeDtypeStruct((P * Hout * Wout,), jnp.float32),
        scratch_types=[
            pltpu.VMEM((IN_STRIP,), jnp.float32),
            pltpu.VMEM((IN_STRIP,), jnp.float32),
            pltpu.VMEM((IN_STRIP,), jnp.int32),
            pltpu.VMEM((IN_STRIP,), jnp.int32),
            pltpu.VMEM((IN_STRIP,), jnp.int32),
            pltpu.VMEM((IN_STRIP,), jnp.int32),
            pltpu.VMEM((OUT_STRIP,), jnp.float32),
            pltpu.VMEM((OUT_STRIP,), jnp.float32),
            pltpu.SemaphoreType.DMA,
            pltpu.SemaphoreType.DMA,
            pltpu.SemaphoreType.DMA,
            pltpu.SemaphoreType.DMA,
            pltpu.SemaphoreType.DMA,
            pltpu.SemaphoreType.DMA,
            pltpu.SemaphoreType.DMA,
            pltpu.SemaphoreType.DMA,
        ],
        compiler_params=pltpu.CompilerParams(needs_layout_passes=False),
    )
    def unpool(in_hbm, idx_hbm, out_hbm, in_v0, in_v1,
               idx_v0, idx_v1, idx_v2, idx_v3, out_v0, out_v1,
               si0, si1, sx0, sx1, sx2, sx3, so0, so1):
        wid = lax.axis_index("s") * NC + lax.axis_index("c")
        base = wid * NSTRIPS
        in_b = (in_v0, in_v1)
        idx_b = (idx_v0, idx_v1, idx_v2, idx_v3)
        out_b = (out_v0, out_v1)
        sin = (si0, si1)
        sidx = (sx0, sx1, sx2, sx3)
        sout = (so0, so1)
        zeros16 = jnp.zeros((16,), jnp.float32)

        def in_copy(ke, b):
            g = base + ke
            return pltpu.make_async_copy(
                in_hbm.at[pl.ds(g * IN_STRIP, IN_STRIP)], in_b[b], sin[b])

        def idx_copy(ke, q):
            g = base + ke
            return pltpu.make_async_copy(
                idx_hbm.at[pl.ds(g * IN_STRIP, IN_STRIP)], idx_b[q], sidx[q])

        def out_copy(ke, b):
            g = base + ke
            return pltpu.make_async_copy(
                out_b[b], out_hbm.at[pl.ds(g * OUT_STRIP, OUT_STRIP)], sout[b])

        in_copy(0, 0).start()
        idx_copy(0, 0).start()
        in_copy(1, 1).start()
        idx_copy(1, 1).start()

        def zero_all(j, _):
            out_v0[pl.ds(j * 16, 16)] = zeros16
            out_v1[pl.ds(j * 16, 16)] = zeros16
            return 0

        lax.fori_loop(0, OUT_STRIP // 16, zero_all, 0, unroll=8)

        def quad_body(kk, _):
            k = kk * 4
            for b in range(4):
                ke = k + b
                ob = b % 2
                q = b
                qprev = (b + 2) % 4

                in_copy(ke, ob).wait()

                @pl.when(ke >= 2)
                def _():
                    out_copy(ke - 2, ob).wait()
                    off_prev = ((ke - 2) % S) * OUT_STRIP

                    def unz_body(i, _):
                        ids = idx_b[qprev][pl.ds(i * 16, 16)] - off_prev
                        plsc.store_scatter(out_b[ob], [ids], zeros16)
                        return 0

                    lax.fori_loop(0, IN_STRIP // 16, unz_body, 0, unroll=8)

                idx_copy(ke, q).wait()
                off = (ke % S) * OUT_STRIP

                def scat_body(i, _):
                    vals = in_b[ob][pl.ds(i * 16, 16)]
                    ids = idx_b[q][pl.ds(i * 16, 16)] - off
                    plsc.store_scatter(out_b[ob], [ids], vals)
                    return 0

                lax.fori_loop(0, IN_STRIP // 16, scat_body, 0, unroll=8)

                out_copy(ke, ob).start()

                @pl.when(ke + 2 < NSTRIPS)
                def _():
                    in_copy(ke + 2, ob).start()
                    idx_copy(ke + 2, qprev).start()
            return 0

        lax.fori_loop(0, NSTRIPS // 4, quad_body, 0)
        out_copy(NSTRIPS - 2, 0).wait()
        out_copy(NSTRIPS - 1, 1).wait()

    out = unpool(in_flat, idx_flat)
    return out.reshape(B, C, Hout, Wout)


# speedup vs baseline: 3.9073x; 3.9073x over previous
import functools

import jax
import jax.numpy as jnp
from jax import lax
from jax.experimental import pallas as pl
from jax.experimental.pallas import tpu as pltpu
from jax.experimental.pallas import tpu_sc as plsc


def kernel(input, indices):
    B, C, H, W = input.shape
    P = B * C
    Hout, Wout = 2 * H, 2 * W

    info = plsc.get_sparse_core_info()
    NC, NS = info.num_cores, info.num_subcores
    NW = NC * NS

    planes_per_w = P // NW
    RH = 48
    S = H // RH
    NSTRIPS = planes_per_w * S

    in3 = input.reshape(P, H, W)
    idx3 = indices.reshape(P, H, W)

    mesh = plsc.VectorSubcoreMesh(core_axis_name="c", subcore_axis_name="s")

    @functools.partial(
        pl.kernel,
        mesh=mesh,
        out_type=jax.ShapeDtypeStruct((P, Hout, Wout), jnp.float32),
        scratch_types=[
            pltpu.VMEM((RH, W), jnp.float32),
            pltpu.VMEM((RH, W), jnp.float32),
            pltpu.VMEM((RH, W), jnp.int32),
            pltpu.VMEM((RH, W), jnp.int32),
            pltpu.VMEM((2 * RH, Wout), jnp.float32),
            pltpu.VMEM((2 * RH, Wout), jnp.float32),
            pltpu.SemaphoreType.DMA,
            pltpu.SemaphoreType.DMA,
            pltpu.SemaphoreType.DMA,
            pltpu.SemaphoreType.DMA,
            pltpu.SemaphoreType.DMA,
            pltpu.SemaphoreType.DMA,
        ],
        compiler_params=pltpu.CompilerParams(needs_layout_passes=False),
    )
    def unpool(in_hbm, idx_hbm, out_hbm, in_v0, in_v1, idx_v0, idx_v1,
               out_v0, out_v1, si0, si1, sx0, sx1, so0, so1):
        wid = lax.axis_index("s") * NC + lax.axis_index("c")
        base = wid * NSTRIPS
        in_b = (in_v0, in_v1)
        idx_b = (idx_v0, idx_v1)
        out_b = (out_v0, out_v1)
        sin = (si0, si1)
        sidx = (sx0, sx1)
        sout = (so0, so1)
        zeros16 = jnp.zeros((16,), jnp.float32)
        lane2 = 2 * lax.iota(jnp.int32, 16)

        def coords(ke):
            g = base + ke
            return g // S, g % S

        def in_copy(ke, b):
            p, s = coords(ke)
            return pltpu.make_async_copy(
                in_hbm.at[p, pl.ds(s * RH, RH)], in_b[b], sin[b])

        def idx_copy(ke, b):
            p, s = coords(ke)
            return pltpu.make_async_copy(
                idx_hbm.at[p, pl.ds(s * RH, RH)], idx_b[b], sidx[b])

        def out_copy(ke, b):
            p, s = coords(ke)
            return pltpu.make_async_copy(
                out_b[b], out_hbm.at[p, pl.ds(s * 2 * RH, 2 * RH)], sout[b])

        in_copy(0, 0).start()
        idx_copy(0, 0).start()
        in_copy(1, 1).start()
        idx_copy(1, 1).start()

        def pair_body(kk, _):
            k = kk * 2
            for b in range(2):
                ke = k + b
                in_copy(ke, b).wait()
                idx_copy(ke, b).wait()

                @pl.when(ke >= 2)
                def _():
                    out_copy(ke - 2, b).wait()

                @plsc.parallel_loop(0, 2 * RH, unroll=2)
                def zero_body(rr):
                    @plsc.parallel_loop(0, Wout, step=16, unroll=8)
                    def _(cc):
                        out_b[b][rr, pl.ds(cc, 16)] = zeros16

                _, s = coords(ke)
                h0 = s * RH

                @plsc.parallel_loop(0, RH, unroll=1)
                def scat_row(r):
                    rowbase = 2 * (h0 + r) * Wout

                    @plsc.parallel_loop(0, W, step=16, unroll=4)
                    def _(c):
                        vals = in_b[b][r, pl.ds(c, 16)]
                        idxv = idx_b[b][r, pl.ds(c, 16)]
                        cb = 2 * c + lane2
                        rel = idxv - (rowbase + cb)
                        dh = (rel >> 7) & 1
                        dw = rel & 1
                        rows = 2 * r + dh
                        cols = cb + dw
                        plsc.store_scatter(out_b[b], [rows, cols], vals)

                out_copy(ke, b).start()

                @pl.when(ke + 2 < NSTRIPS)
                def _():
                    in_copy(ke + 2, b).start()
                    idx_copy(ke + 2, b).start()
            return 0

        lax.fori_loop(0, NSTRIPS // 2, pair_body, 0)
        out_copy(NSTRIPS - 2, 0).wait()
        out_copy(NSTRIPS - 1, 1).wait()

    out = unpool(in3, idx3)
    return out.reshape(B, C, Hout, Wout)


# speedup vs baseline: 4.0387x; 1.0336x over previous
import functools

import jax
import jax.numpy as jnp
from jax import lax
from jax.experimental import pallas as pl
from jax.experimental.pallas import tpu as pltpu
from jax.experimental.pallas import tpu_sc as plsc


def kernel(input, indices):
    B, C, H, W = input.shape
    P = B * C
    Hout, Wout = 2 * H, 2 * W

    info = plsc.get_sparse_core_info()
    NC, NS = info.num_cores, info.num_subcores
    NW = NC * NS

    planes_per_w = P // NW
    RH = 48
    S = H // RH
    NSTRIPS = planes_per_w * S

    in3 = input.reshape(P, H, W)
    idx3 = indices.reshape(P, H, W)

    mesh = plsc.VectorSubcoreMesh(core_axis_name="c", subcore_axis_name="s")

    @functools.partial(
        pl.kernel,
        mesh=mesh,
        out_type=jax.ShapeDtypeStruct((P, Hout, Wout), jnp.float32),
        scratch_types=[
            pltpu.VMEM((RH, W), jnp.float32),
            pltpu.VMEM((RH, W), jnp.float32),
            pltpu.VMEM((RH, W), jnp.int32),
            pltpu.VMEM((RH, W), jnp.int32),
            pltpu.VMEM((2 * RH, Wout), jnp.float32),
            pltpu.VMEM((2 * RH, Wout), jnp.float32),
            pltpu.SemaphoreType.DMA,
            pltpu.SemaphoreType.DMA,
            pltpu.SemaphoreType.DMA,
            pltpu.SemaphoreType.DMA,
            pltpu.SemaphoreType.DMA,
            pltpu.SemaphoreType.DMA,
        ],
        compiler_params=pltpu.CompilerParams(needs_layout_passes=False),
    )
    def unpool(in_hbm, idx_hbm, out_hbm, in_v0, in_v1, idx_v0, idx_v1,
               out_v0, out_v1, si0, si1, sx0, sx1, so0, so1):
        wid = lax.axis_index("s") * NC + lax.axis_index("c")
        base = wid * NSTRIPS
        in_b = (in_v0, in_v1)
        idx_b = (idx_v0, idx_v1)
        out_b = (out_v0, out_v1)
        sin = (si0, si1)
        sidx = (sx0, sx1)
        sout = (so0, so1)
        zeros16 = jnp.zeros((16,), jnp.float32)
        lane2 = 2 * lax.iota(jnp.int32, 16)

        def coords(ke):
            g = base + ke
            return g // S, g % S

        def in_copy(ke, b):
            p, s = coords(ke)
            return pltpu.make_async_copy(
                in_hbm.at[p, pl.ds(s * RH, RH)], in_b[b], sin[b])

        def idx_copy(ke, b):
            p, s = coords(ke)
            return pltpu.make_async_copy(
                idx_hbm.at[p, pl.ds(s * RH, RH)], idx_b[b], sidx[b])

        def out_copy(ke, b):
            p, s = coords(ke)
            return pltpu.make_async_copy(
                out_b[b], out_hbm.at[p, pl.ds(s * 2 * RH, 2 * RH)], sout[b])

        in_copy(0, 0).start()
        idx_copy(0, 0).start()
        in_copy(1, 1).start()
        idx_copy(1, 1).start()

        def pair_body(kk, _):
            k = kk * 2
            for b in range(2):
                ke = k + b
                in_copy(ke, b).wait()
                idx_copy(ke, b).wait()

                @pl.when(ke >= 2)
                def _():
                    out_copy(ke - 2, b).wait()

                @plsc.parallel_loop(0, 2 * RH, unroll=4)
                def zero_body(rr):
                    @plsc.parallel_loop(0, Wout, step=16, unroll=8)
                    def _(cc):
                        out_b[b][rr, pl.ds(cc, 16)] = zeros16

                _, s = coords(ke)
                h0 = s * RH

                @plsc.parallel_loop(0, RH, unroll=1)
                def scat_row(r):
                    rowbase = 2 * (h0 + r) * Wout

                    @plsc.parallel_loop(0, W, step=16, unroll=6)
                    def _(c):
                        vals = in_b[b][r, pl.ds(c, 16)]
                        idxv = idx_b[b][r, pl.ds(c, 16)]
                        cb = 2 * c + lane2
                        rel = idxv - (rowbase + cb)
                        dh = (rel >> 7) & 1
                        dw = rel & 1
                        rows = 2 * r + dh
                        cols = cb + dw
                        plsc.store_scatter(out_b[b], [rows, cols], vals)

                out_copy(ke, b).start()

                @pl.when(ke + 2 < NSTRIPS)
                def _():
                    in_copy(ke + 2, b).start()
                    idx_copy(ke + 2, b).start()
            return 0

        lax.fori_loop(0, NSTRIPS // 2, pair_body, 0)
        out_copy(NSTRIPS - 2, 0).wait()
        out_copy(NSTRIPS - 1, 1).wait()

    out = unpool(in3, idx3)
    return out.reshape(B, C, Hout, Wout)
